# baseline (speedup 1.0000x reference)
import jax
import jax.numpy as jnp
from jax import lax
from jax.experimental import pallas as pl
from jax.experimental.pallas import tpu as pltpu
from jax.experimental.pallas import tpu_sc as plsc

B = 16384
F = 26
D = 16
NW = 32
E = B // NW
C = 128
NCH = E // C
RPC = F * C
IW = 128
NG = RPC // IW

NR = 1000012
GW = 512
TG = NR // GW
TAILR = NR - TG * GW
TGQ, TGR = divmod(TG, NW)


NB = 4


def _tr_body(emb_t, tail, out, b0, b1, b2, b3, o0, o1, o2, o3, tvbuf, sems):
    bufs = [b0, b1, b2, b3]
    obufs = [o0, o1, o2, o3]
    wid = lax.axis_index("c") * 16 + lax.axis_index("s")
    baseg = wid * TGQ + jnp.minimum(wid, TGR)
    cntg = TGQ + (wid < TGR).astype(jnp.int32)
    lastg = baseg + cntg
    lane16 = lax.iota(jnp.int32, 16) * 16

    def fire_in(g, j):
        pltpu.async_copy(emb_t.at[:, pl.ds(g * GW, GW)], bufs[j],
                         sems.at[j])

    def wait_in(j):
        pltpu.make_async_copy(emb_t.at[:, pl.ds(0, GW)], bufs[j],
                              sems.at[j]).wait()

    def fire_out(g, j):
        pltpu.async_copy(obufs[j], out.at[pl.ds(g * (GW * D), GW * D)],
                         sems.at[NB + j])

    def wait_out(j):
        pltpu.make_async_copy(obufs[j], out.at[pl.ds(0, GW * D)],
                              sems.at[NB + j]).wait()

    def compute(j):
        buf = bufs[j]
        obuf = obufs[j]

        @pl.loop(0, GW // 16, unroll=2)
        def _c16(c):
            vec = lane16 + c * 256
            for d in range(D):
                v = buf[d, pl.ds(c * 16, 16)]
                plsc.store_scatter(obuf, [vec + d], v)

    for j in range(NB):
        @pl.when(baseg + j < lastg)
        def _prime():
            fire_in(baseg + j, j)

    @pl.loop(0, (cntg + NB - 1) // NB)
    def _quad(q):
        for j in range(NB):
            g = baseg + NB * q + j

            @pl.when(g < lastg)
            def _slot():
                wait_in(j)

                @pl.when(q > 0)
                def _wo():
                    wait_out(j)

                compute(j)
                fire_out(g, j)

                @pl.when(g + NB < lastg)
                def _refill():
                    fire_in(g + NB, j)

    for j in range(NB):
        wait_out(j)

    @pl.when(wid == 0)
    def _tail():
        pltpu.sync_copy(tail, tvbuf)
        pltpu.sync_copy(tvbuf, out.at[pl.ds(TG * GW * D, TAILR * D)])


def _fm_body(x_r, emb, fc, wv, bv, out, idx2, rows2, fcv2, outv, tbuf, pv, sems):
    wid = lax.axis_index("c") * 16 + lax.axis_index("s")

    pltpu.sync_copy(wv, pv.at[0])
    pltpu.sync_copy(bv, pv.at[1])

    def fire(g, b):
        row0 = (wid * NCH + g) * NG
        pltpu.sync_copy(x_r.at[pl.ds(row0, NG)], idx2.at[b])
        for r in range(NG):
            pltpu.async_copy(emb.at[idx2.at[b, r]],
                             rows2.at[b, pl.ds(r * IW, IW)], sems.at[b])
            pltpu.async_copy(fc.at[idx2.at[b, r]],
                             fcv2.at[b, pl.ds(r * IW, IW)], sems.at[b])

    def drain(b):
        for r in range(NG):
            pltpu.make_async_copy(emb.at[idx2.at[b, r]],
                                  rows2.at[b, pl.ds(r * IW, IW)],
                                  sems.at[b]).wait()
            pltpu.make_async_copy(fc.at[idx2.at[b, r]],
                                  fcv2.at[b, pl.ds(r * IW, IW)],
                                  sems.at[b]).wait()

    lane = lax.iota(jnp.int32, 16)
    lane16 = lane * 16
    mask10 = (lane < (F - 16)).astype(jnp.float32)

    def compute(g, b):
        wvec = pv[0, :]
        bvec = pv[1, :]
        for grp in range(C // 16):
            @pl.loop(0, 16)
            def _elem(i):
                e = grp * 16 + i
                r0 = e * F
                s = jnp.zeros((16,), jnp.float32)
                q = jnp.zeros((16,), jnp.float32)
                for f in range(F):
                    v = rows2[b, r0 + f, :]
                    s = s + v
                    q = q + v * v
                f1 = fcv2[b, pl.ds(r0, 16)]
                f2 = fcv2[b, pl.ds(r0 + 16, 16)]
                t = 0.5 * (s * s - q) + wvec * (f1 + f2 * mask10)
                tbuf[pl.ds(i * 16, 16)] = t

            acc = jnp.zeros((16,), jnp.float32)
            for dcol in range(16):
                acc = acc + plsc.load_gather(tbuf, [lane16 + dcol])
            tot = acc + bvec
            outv[pl.ds(grp * 16, 16)] = 1.0 / (1.0 + jnp.exp(-tot))
        pltpu.sync_copy(outv, out.at[pl.ds(wid * E + g * C, C)])

    fire(0, 0)

    @pl.loop(0, NCH, step=2)
    def _chunks(gg):
        fire(gg + 1, 1)
        drain(0)
        compute(gg, 0)

        @pl.when(gg + 2 < NCH)
        def _refill():
            fire(gg + 2, 0)

        drain(1)
        compute(gg + 1, 1)


def kernel(x, emb_table, fc_table, lin_w, lin_b):
    mesh = plsc.VectorSubcoreMesh(core_axis_name="c", subcore_axis_name="s")

    emb_t = emb_table.T
    tail = emb_table[TG * GW:, :].reshape(-1)
    emb_lin = pl.kernel(
        _tr_body,
        out_type=jax.ShapeDtypeStruct((NR * D,), jnp.float32),
        mesh=mesh,
        compiler_params=pltpu.CompilerParams(needs_layout_passes=False,
                                             use_tc_tiling_on_sc=True),
        scratch_types=(
            [pltpu.VMEM((D, GW), jnp.float32) for _ in range(NB)]
            + [pltpu.VMEM((GW * D,), jnp.float32) for _ in range(NB)]
            + [pltpu.VMEM((TAILR * D,), jnp.float32),
               pltpu.SemaphoreType.DMA((2 * NB,))]
        ),
    )(emb_t, tail)
    emb2 = emb_lin.reshape(NR, D)

    x_r = x.astype(jnp.int32).reshape(B * F // IW, IW)
    fc_flat = fc_table.reshape(-1)
    wv = jnp.broadcast_to(lin_w.reshape(()), (16,)).astype(jnp.float32)
    bv = jnp.broadcast_to(lin_b.reshape(()), (16,)).astype(jnp.float32)

    out = pl.kernel(
        _fm_body,
        out_type=jax.ShapeDtypeStruct((B,), jnp.float32),
        mesh=mesh,
        compiler_params=pltpu.CompilerParams(needs_layout_passes=False,
                                             use_tc_tiling_on_sc=False),
        scratch_types=[
            pltpu.VMEM((2, NG, IW), jnp.int32),
            pltpu.VMEM((2, RPC, D), jnp.float32),
            pltpu.VMEM((2, RPC + 16), jnp.float32),
            pltpu.VMEM((C,), jnp.float32),
            pltpu.VMEM((256,), jnp.float32),
            pltpu.VMEM((2, 16), jnp.float32),
            pltpu.SemaphoreType.DMA((2,)),
        ],
    )(x_r, emb2, fc_flat, wv, bv)
    return out.reshape(B, 1)


# speedup vs baseline: 1.0559x; 1.0559x over previous
import jax
import jax.numpy as jnp
from jax import lax
from jax.experimental import pallas as pl
from jax.experimental.pallas import tpu as pltpu
from jax.experimental.pallas import tpu_sc as plsc

B = 16384
F = 26
D = 16
NW = 32
E = B // NW
C = 64
NCH = E // C
RPC = F * C
IW = 128
NG = RPC // IW

NR = 1000012
GW = 512
TG = NR // GW
TAILR = NR - TG * GW
TGQ, TGR = divmod(TG, NW)


NB = 2


def _tr_body(emb_t, tail, out, b0, b1, o0, o1, tvbuf, sems):
    bufs = [b0, b1]
    obufs = [o0, o1]
    wid = lax.axis_index("c") * 16 + lax.axis_index("s")
    baseg = wid * TGQ + jnp.minimum(wid, TGR)
    cntg = TGQ + (wid < TGR).astype(jnp.int32)
    lastg = baseg + cntg
    lane16 = lax.iota(jnp.int32, 16) * 16

    def fire_in(g, j):
        pltpu.async_copy(emb_t.at[:, pl.ds(g * GW, GW)], bufs[j],
                         sems.at[j])

    def wait_in(j):
        pltpu.make_async_copy(emb_t.at[:, pl.ds(0, GW)], bufs[j],
                              sems.at[j]).wait()

    def fire_out(g, j):
        pltpu.async_copy(obufs[j], out.at[pl.ds(g * (GW * D), GW * D)],
                         sems.at[NB + j])

    def wait_out(j):
        pltpu.make_async_copy(obufs[j], out.at[pl.ds(0, GW * D)],
                              sems.at[NB + j]).wait()

    def compute(j):
        buf = bufs[j]
        obuf = obufs[j]

        @pl.loop(0, GW // 16)
        def _c16(c):
            vec = lane16 + c * 256
            for d in range(D):
                v = buf[d, pl.ds(c * 16, 16)]
                plsc.store_scatter(obuf, [vec + d], v)

    for j in range(NB):
        @pl.when(baseg + j < lastg)
        def _prime():
            fire_in(baseg + j, j)

    @pl.loop(0, (cntg + NB - 1) // NB)
    def _quad(q):
        for j in range(NB):
            g = baseg + NB * q + j

            @pl.when(g < lastg)
            def _slot():
                wait_in(j)

                @pl.when(q > 0)
                def _wo():
                    wait_out(j)

                compute(j)
                fire_out(g, j)

                @pl.when(g + NB < lastg)
                def _refill():
                    fire_in(g + NB, j)

    for j in range(NB):
        wait_out(j)

    @pl.when(wid == 0)
    def _tail():
        pltpu.sync_copy(tail, tvbuf)
        pltpu.sync_copy(tvbuf, out.at[pl.ds(TG * GW * D, TAILR * D)])


def _fm_body(x_r, emb, fc, wv, bv, out, idx2, rows2, fcv2, outv, tbuf, pv, sems):
    wid = lax.axis_index("c") * 16 + lax.axis_index("s")

    pltpu.sync_copy(wv, pv.at[0])
    pltpu.sync_copy(bv, pv.at[1])

    def fire(g, b):
        row0 = (wid * NCH + g) * NG
        pltpu.sync_copy(x_r.at[pl.ds(row0, NG)], idx2.at[b])
        for r in range(NG):
            pltpu.async_copy(emb.at[idx2.at[b, r]],
                             rows2.at[b, pl.ds(r * IW, IW)], sems.at[b])
            pltpu.async_copy(fc.at[idx2.at[b, r]],
                             fcv2.at[b, pl.ds(r * IW, IW)], sems.at[b])

    def drain(b):
        for r in range(NG):
            pltpu.make_async_copy(emb.at[idx2.at[b, r]],
                                  rows2.at[b, pl.ds(r * IW, IW)],
                                  sems.at[b]).wait()
            pltpu.make_async_copy(fc.at[idx2.at[b, r]],
                                  fcv2.at[b, pl.ds(r * IW, IW)],
                                  sems.at[b]).wait()

    lane = lax.iota(jnp.int32, 16)
    lane16 = lane * 16
    mask10 = (lane < (F - 16)).astype(jnp.float32)

    def compute(g, b):
        wvec = pv[0, :]
        bvec = pv[1, :]
        for grp in range(C // 16):
            @pl.loop(0, 16)
            def _elem(i):
                e = grp * 16 + i
                r0 = e * F
                s = jnp.zeros((16,), jnp.float32)
                q = jnp.zeros((16,), jnp.float32)
                for f in range(F):
                    v = rows2[b, r0 + f, :]
                    s = s + v
                    q = q + v * v
                f1 = fcv2[b, pl.ds(r0, 16)]
                f2 = fcv2[b, pl.ds(r0 + 16, 16)]
                t = 0.5 * (s * s - q) + wvec * (f1 + f2 * mask10)
                tbuf[pl.ds(i * 16, 16)] = t

            acc = jnp.zeros((16,), jnp.float32)
            for dcol in range(16):
                acc = acc + plsc.load_gather(tbuf, [lane16 + dcol])
            tot = acc + bvec
            outv[pl.ds(grp * 16, 16)] = 1.0 / (1.0 + jnp.exp(-tot))
        pltpu.sync_copy(outv, out.at[pl.ds(wid * E + g * C, C)])

    fire(0, 0)

    @pl.loop(0, NCH, step=2)
    def _chunks(gg):
        fire(gg + 1, 1)
        drain(0)
        compute(gg, 0)

        @pl.when(gg + 2 < NCH)
        def _refill():
            fire(gg + 2, 0)

        drain(1)
        compute(gg + 1, 1)


def kernel(x, emb_table, fc_table, lin_w, lin_b):
    mesh = plsc.VectorSubcoreMesh(core_axis_name="c", subcore_axis_name="s")

    emb_t = emb_table.T
    tail = emb_table[TG * GW:, :].reshape(-1)
    emb_lin = pl.kernel(
        _tr_body,
        out_type=jax.ShapeDtypeStruct((NR * D,), jnp.float32),
        mesh=mesh,
        compiler_params=pltpu.CompilerParams(needs_layout_passes=False,
                                             use_tc_tiling_on_sc=True),
        scratch_types=(
            [pltpu.VMEM((D, GW), jnp.float32) for _ in range(NB)]
            + [pltpu.VMEM((GW * D,), jnp.float32) for _ in range(NB)]
            + [pltpu.VMEM((TAILR * D,), jnp.float32),
               pltpu.SemaphoreType.DMA((2 * NB,))]
        ),
    )(emb_t, tail)
    emb2 = emb_lin.reshape(NR, D)

    x_r = x.astype(jnp.int32).reshape(B * F // IW, IW)
    fc_flat = fc_table.reshape(-1)
    wv = jnp.broadcast_to(lin_w.reshape(()), (16,)).astype(jnp.float32)
    bv = jnp.broadcast_to(lin_b.reshape(()), (16,)).astype(jnp.float32)

    out = pl.kernel(
        _fm_body,
        out_type=jax.ShapeDtypeStruct((B,), jnp.float32),
        mesh=mesh,
        compiler_params=pltpu.CompilerParams(needs_layout_passes=False,
                                             use_tc_tiling_on_sc=False),
        scratch_types=[
            pltpu.VMEM((2, NG, IW), jnp.int32),
            pltpu.VMEM((2, RPC, D), jnp.float32),
            pltpu.VMEM((2, RPC + 16), jnp.float32),
            pltpu.VMEM((C,), jnp.float32),
            pltpu.VMEM((256,), jnp.float32),
            pltpu.VMEM((2, 16), jnp.float32),
            pltpu.SemaphoreType.DMA((2,)),
        ],
    )(x_r, emb2, fc_flat, wv, bv)
    return out.reshape(B, 1)


# speedup vs baseline: 1.0585x; 1.0025x over previous
import jax
import jax.numpy as jnp
from jax import lax
from jax.experimental import pallas as pl
from jax.experimental.pallas import tpu as pltpu
from jax.experimental.pallas import tpu_sc as plsc

B = 16384
F = 26
D = 16
NW = 32
E = B // NW
C = 64
NCH = E // C
RPC = F * C
IW = 128
NG = RPC // IW

NR = 1000012
GW = 512
TG = NR // GW
TAILR = NR - TG * GW
TGQ, TGR = divmod(TG, NW)


NB = 2


def _tr_body(emb_t, tail, out, b0, b1, o0, o1, tvbuf, sems):
    bufs = [b0, b1]
    obufs = [o0, o1]
    wid = lax.axis_index("c") * 16 + lax.axis_index("s")
    baseg = wid * TGQ + jnp.minimum(wid, TGR)
    cntg = TGQ + (wid < TGR).astype(jnp.int32)
    lastg = baseg + cntg
    lane16 = lax.iota(jnp.int32, 16) * 16

    def fire_in(g, j):
        pltpu.async_copy(emb_t.at[:, pl.ds(g * GW, GW)], bufs[j],
                         sems.at[j])

    def wait_in(j):
        pltpu.make_async_copy(emb_t.at[:, pl.ds(0, GW)], bufs[j],
                              sems.at[j]).wait()

    def fire_out(g, j):
        pltpu.async_copy(obufs[j], out.at[pl.ds(g * (GW * D), GW * D)],
                         sems.at[NB + j])

    def wait_out(j):
        pltpu.make_async_copy(obufs[j], out.at[pl.ds(0, GW * D)],
                              sems.at[NB + j]).wait()

    def compute(j):
        buf = bufs[j]
        obuf = obufs[j]

        @pl.loop(0, GW // 16)
        def _c16(c):
            vec = lane16 + c * 256
            for d in range(D):
                v = buf[d, pl.ds(c * 16, 16)]
                plsc.store_scatter(obuf, [vec + d], v)

    for j in range(NB):
        @pl.when(baseg + j < lastg)
        def _prime():
            fire_in(baseg + j, j)

    @pl.loop(0, (cntg + NB - 1) // NB)
    def _quad(q):
        for j in range(NB):
            g = baseg + NB * q + j

            @pl.when(g < lastg)
            def _slot():
                wait_in(j)

                @pl.when(q > 0)
                def _wo():
                    wait_out(j)

                compute(j)
                fire_out(g, j)

                @pl.when(g + NB < lastg)
                def _refill():
                    fire_in(g + NB, j)

    for j in range(NB):
        wait_out(j)

    @pl.when(wid == 0)
    def _tail():
        pltpu.sync_copy(tail, tvbuf)
        pltpu.sync_copy(tvbuf, out.at[pl.ds(TG * GW * D, TAILR * D)])


def _fm_body(x_r, emb, fc, wv, bv, out, idx2, rows2, fcv2, outv, tbuf, pv, sems):
    wid = lax.axis_index("c") * 16 + lax.axis_index("s")

    pltpu.sync_copy(wv, pv.at[0])
    pltpu.sync_copy(bv, pv.at[1])

    def fire(g, b):
        row0 = (wid * NCH + g) * NG
        pltpu.sync_copy(x_r.at[pl.ds(row0, NG)], idx2.at[b])
        for r in range(NG):
            pltpu.async_copy(emb.at[idx2.at[b, r]],
                             rows2.at[b, pl.ds(r * IW, IW)], sems.at[b])
            pltpu.async_copy(fc.at[idx2.at[b, r]],
                             fcv2.at[b, pl.ds(r * IW, IW)], sems.at[b])

    def drain(b):
        pltpu.make_async_copy(emb.at[pl.ds(0, RPC)], rows2.at[b],
                              sems.at[b]).wait()
        pltpu.make_async_copy(fc.at[pl.ds(0, RPC)],
                              fcv2.at[b, pl.ds(0, RPC)], sems.at[b]).wait()

    lane = lax.iota(jnp.int32, 16)
    lane16 = lane * 16
    mask10 = (lane < (F - 16)).astype(jnp.float32)

    def compute(g, b):
        wvec = pv[0, :]
        bvec = pv[1, :]
        for grp in range(C // 16):
            @pl.loop(0, 16)
            def _elem(i):
                e = grp * 16 + i
                r0 = e * F
                s = jnp.zeros((16,), jnp.float32)
                q = jnp.zeros((16,), jnp.float32)
                for f in range(F):
                    v = rows2[b, r0 + f, :]
                    s = s + v
                    q = q + v * v
                f1 = fcv2[b, pl.ds(r0, 16)]
                f2 = fcv2[b, pl.ds(r0 + 16, 16)]
                t = 0.5 * (s * s - q) + wvec * (f1 + f2 * mask10)
                tbuf[pl.ds(i * 16, 16)] = t

            acc = jnp.zeros((16,), jnp.float32)
            for dcol in range(16):
                acc = acc + plsc.load_gather(tbuf, [lane16 + dcol])
            tot = acc + bvec
            outv[pl.ds(grp * 16, 16)] = 1.0 / (1.0 + jnp.exp(-tot))
        pltpu.sync_copy(outv, out.at[pl.ds(wid * E + g * C, C)])

    fire(0, 0)

    @pl.loop(0, NCH, step=2)
    def _chunks(gg):
        fire(gg + 1, 1)
        drain(0)
        compute(gg, 0)

        @pl.when(gg + 2 < NCH)
        def _refill():
            fire(gg + 2, 0)

        drain(1)
        compute(gg + 1, 1)


def kernel(x, emb_table, fc_table, lin_w, lin_b):
    mesh = plsc.VectorSubcoreMesh(core_axis_name="c", subcore_axis_name="s")

    emb_t = emb_table.T
    tail = emb_table[TG * GW:, :].reshape(-1)
    emb_lin = pl.kernel(
        _tr_body,
        out_type=jax.ShapeDtypeStruct((NR * D,), jnp.float32),
        mesh=mesh,
        compiler_params=pltpu.CompilerParams(needs_layout_passes=False,
                                             use_tc_tiling_on_sc=True),
        scratch_types=(
            [pltpu.VMEM((D, GW), jnp.float32) for _ in range(NB)]
            + [pltpu.VMEM((GW * D,), jnp.float32) for _ in range(NB)]
            + [pltpu.VMEM((TAILR * D,), jnp.float32),
               pltpu.SemaphoreType.DMA((2 * NB,))]
        ),
    )(emb_t, tail)
    emb2 = emb_lin.reshape(NR, D)

    x_r = x.astype(jnp.int32).reshape(B * F // IW, IW)
    fc_flat = fc_table.reshape(-1)
    wv = jnp.broadcast_to(lin_w.reshape(()), (16,)).astype(jnp.float32)
    bv = jnp.broadcast_to(lin_b.reshape(()), (16,)).astype(jnp.float32)

    out = pl.kernel(
        _fm_body,
        out_type=jax.ShapeDtypeStruct((B,), jnp.float32),
        mesh=mesh,
        compiler_params=pltpu.CompilerParams(needs_layout_passes=False,
                                             use_tc_tiling_on_sc=False),
        scratch_types=[
            pltpu.VMEM((2, NG, IW), jnp.int32),
            pltpu.VMEM((2, RPC, D), jnp.float32),
            pltpu.VMEM((2, RPC + 16), jnp.float32),
            pltpu.VMEM((C,), jnp.float32),
            pltpu.VMEM((256,), jnp.float32),
            pltpu.VMEM((2, 16), jnp.float32),
            pltpu.SemaphoreType.DMA((2,)),
        ],
    )(x_r, emb2, fc_flat, wv, bv)
    return out.reshape(B, 1)


# speedup vs baseline: 1.4605x; 1.3797x over previous
import jax
import jax.numpy as jnp
from jax import lax
from jax.experimental import pallas as pl
from jax.experimental.pallas import tpu as pltpu
from jax.experimental.pallas import tpu_sc as plsc

B = 16384
F = 26
D = 16
NW = 32
E = B // NW
C = 64
NCH = E // C
RPC = F * C
IW = 128
NG = RPC // IW

NR = 1000012
GW = 512
TG = NR // GW
TAILR = NR - TG * GW
TGQ, TGR = divmod(TG, NW)


NB = 2


def _tr_body(emb_t, tail, out, b0, b1, o0, o1, tvbuf, sems):
    bufs = [b0, b1]
    obufs = [o0, o1]
    wid = lax.axis_index("c") * 16 + lax.axis_index("s")
    baseg = wid * TGQ + jnp.minimum(wid, TGR)
    cntg = TGQ + (wid < TGR).astype(jnp.int32)
    lastg = baseg + cntg
    lane16 = lax.iota(jnp.int32, 16) * 16

    def fire_in(g, j):
        pltpu.async_copy(emb_t.at[:, pl.ds(g * GW, GW)], bufs[j],
                         sems.at[j])

    def wait_in(j):
        pltpu.make_async_copy(emb_t.at[:, pl.ds(0, GW)], bufs[j],
                              sems.at[j]).wait()

    def fire_out(g, j):
        pltpu.async_copy(obufs[j], out.at[pl.ds(g * (GW * D), GW * D)],
                         sems.at[NB + j])

    def wait_out(j):
        pltpu.make_async_copy(obufs[j], out.at[pl.ds(0, GW * D)],
                              sems.at[NB + j]).wait()

    def compute(j):
        buf = bufs[j]
        obuf = obufs[j]

        @plsc.parallel_loop(0, GW // 16)
        def _c16(c):
            vec = lane16 + c * 256
            for d in range(D):
                v = buf[d, pl.ds(c * 16, 16)]
                plsc.store_scatter(obuf, [vec + d], v)

    for j in range(NB):
        @pl.when(baseg + j < lastg)
        def _prime():
            fire_in(baseg + j, j)

    @pl.loop(0, (cntg + NB - 1) // NB)
    def _quad(q):
        for j in range(NB):
            g = baseg + NB * q + j

            @pl.when(g < lastg)
            def _slot():
                wait_in(j)

                @pl.when(q > 0)
                def _wo():
                    wait_out(j)

                compute(j)
                fire_out(g, j)

                @pl.when(g + NB < lastg)
                def _refill():
                    fire_in(g + NB, j)

    for j in range(NB):
        wait_out(j)

    @pl.when(wid == 0)
    def _tail():
        pltpu.sync_copy(tail, tvbuf)
        pltpu.sync_copy(tvbuf, out.at[pl.ds(TG * GW * D, TAILR * D)])


def _fm_body(x_r, emb, fc, wv, bv, out, idx2, rows2, fcv2, outv, tbuf, pv, sems):
    wid = lax.axis_index("c") * 16 + lax.axis_index("s")

    pltpu.sync_copy(wv, pv.at[0])
    pltpu.sync_copy(bv, pv.at[1])

    def fire(g, b):
        row0 = (wid * NCH + g) * NG
        pltpu.sync_copy(x_r.at[pl.ds(row0, NG)], idx2.at[b])
        for r in range(NG):
            pltpu.async_copy(emb.at[idx2.at[b, r]],
                             rows2.at[b, pl.ds(r * IW, IW)], sems.at[b])
            pltpu.async_copy(fc.at[idx2.at[b, r]],
                             fcv2.at[b, pl.ds(r * IW, IW)], sems.at[b])

    def drain(b):
        pltpu.make_async_copy(emb.at[pl.ds(0, RPC)], rows2.at[b],
                              sems.at[b]).wait()
        pltpu.make_async_copy(fc.at[pl.ds(0, RPC)],
                              fcv2.at[b, pl.ds(0, RPC)], sems.at[b]).wait()

    lane = lax.iota(jnp.int32, 16)
    lane16 = lane * 16
    mask10 = (lane < (F - 16)).astype(jnp.float32)

    def compute(g, b):
        wvec = pv[0, :]
        bvec = pv[1, :]
        for grp in range(C // 16):
            @pl.loop(0, 16)
            def _elem(i):
                e = grp * 16 + i
                r0 = e * F
                s = jnp.zeros((16,), jnp.float32)
                q = jnp.zeros((16,), jnp.float32)
                for f in range(F):
                    v = rows2[b, r0 + f, :]
                    s = s + v
                    q = q + v * v
                f1 = fcv2[b, pl.ds(r0, 16)]
                f2 = fcv2[b, pl.ds(r0 + 16, 16)]
                t = 0.5 * (s * s - q) + wvec * (f1 + f2 * mask10)
                tbuf[pl.ds(i * 16, 16)] = t

            acc = jnp.zeros((16,), jnp.float32)
            for dcol in range(16):
                acc = acc + plsc.load_gather(tbuf, [lane16 + dcol])
            tot = acc + bvec
            outv[pl.ds(grp * 16, 16)] = 1.0 / (1.0 + jnp.exp(-tot))
        pltpu.sync_copy(outv, out.at[pl.ds(wid * E + g * C, C)])

    fire(0, 0)

    @pl.loop(0, NCH, step=2)
    def _chunks(gg):
        fire(gg + 1, 1)
        drain(0)
        compute(gg, 0)

        @pl.when(gg + 2 < NCH)
        def _refill():
            fire(gg + 2, 0)

        drain(1)
        compute(gg + 1, 1)


def kernel(x, emb_table, fc_table, lin_w, lin_b):
    mesh = plsc.VectorSubcoreMesh(core_axis_name="c", subcore_axis_name="s")

    emb_t = emb_table.T
    tail = emb_table[TG * GW:, :].reshape(-1)
    emb_lin = pl.kernel(
        _tr_body,
        out_type=jax.ShapeDtypeStruct((NR * D,), jnp.float32),
        mesh=mesh,
        compiler_params=pltpu.CompilerParams(needs_layout_passes=False,
                                             use_tc_tiling_on_sc=True),
        scratch_types=(
            [pltpu.VMEM((D, GW), jnp.float32) for _ in range(NB)]
            + [pltpu.VMEM((GW * D,), jnp.float32) for _ in range(NB)]
            + [pltpu.VMEM((TAILR * D,), jnp.float32),
               pltpu.SemaphoreType.DMA((2 * NB,))]
        ),
    )(emb_t, tail)
    emb2 = emb_lin.reshape(NR, D)

    x_r = x.astype(jnp.int32).reshape(B * F // IW, IW)
    fc_flat = fc_table.reshape(-1)
    wv = jnp.broadcast_to(lin_w.reshape(()), (16,)).astype(jnp.float32)
    bv = jnp.broadcast_to(lin_b.reshape(()), (16,)).astype(jnp.float32)

    out = pl.kernel(
        _fm_body,
        out_type=jax.ShapeDtypeStruct((B,), jnp.float32),
        mesh=mesh,
        compiler_params=pltpu.CompilerParams(needs_layout_passes=False,
                                             use_tc_tiling_on_sc=False),
        scratch_types=[
            pltpu.VMEM((2, NG, IW), jnp.int32),
            pltpu.VMEM((2, RPC, D), jnp.float32),
            pltpu.VMEM((2, RPC + 16), jnp.float32),
            pltpu.VMEM((C,), jnp.float32),
            pltpu.VMEM((256,), jnp.float32),
            pltpu.VMEM((2, 16), jnp.float32),
            pltpu.SemaphoreType.DMA((2,)),
        ],
    )(x_r, emb2, fc_flat, wv, bv)
    return out.reshape(B, 1)


# speedup vs baseline: 1.4920x; 1.0216x over previous
import jax
import jax.numpy as jnp
from jax import lax
from jax.experimental import pallas as pl
from jax.experimental.pallas import tpu as pltpu
from jax.experimental.pallas import tpu_sc as plsc

B = 16384
F = 26
D = 16
NW = 32
E = B // NW
C = 64
NCH = E // C
RPC = F * C
IW = 128
NG = RPC // IW

NR = 1000012
GW = 512
TG = NR // GW
TAILR = NR - TG * GW
TGQ, TGR = divmod(TG, NW)


NB = 2


def _tr_body(emb_t, tail, out, b0, b1, o0, o1, tvbuf, sems):
    bufs = [b0, b1]
    obufs = [o0, o1]
    wid = lax.axis_index("c") * 16 + lax.axis_index("s")
    baseg = wid * TGQ + jnp.minimum(wid, TGR)
    cntg = TGQ + (wid < TGR).astype(jnp.int32)
    lastg = baseg + cntg
    lane16 = lax.iota(jnp.int32, 16) * 16

    def fire_in(g, j):
        pltpu.async_copy(emb_t.at[:, pl.ds(g * GW, GW)], bufs[j],
                         sems.at[j])

    def wait_in(j):
        pltpu.make_async_copy(emb_t.at[:, pl.ds(0, GW)], bufs[j],
                              sems.at[j]).wait()

    def fire_out(g, j):
        pltpu.async_copy(obufs[j], out.at[pl.ds(g * (GW * D), GW * D)],
                         sems.at[NB + j])

    def wait_out(j):
        pltpu.make_async_copy(obufs[j], out.at[pl.ds(0, GW * D)],
                              sems.at[NB + j]).wait()

    def compute(j):
        buf = bufs[j]
        obuf = obufs[j]

        @plsc.parallel_loop(0, GW // 16)
        def _c16(c):
            vec = lane16 + c * 256
            for d in range(D):
                v = buf[d, pl.ds(c * 16, 16)]
                plsc.store_scatter(obuf, [vec + d], v)

    for j in range(NB):
        @pl.when(baseg + j < lastg)
        def _prime():
            fire_in(baseg + j, j)

    @pl.loop(0, (cntg + NB - 1) // NB)
    def _quad(q):
        for j in range(NB):
            g = baseg + NB * q + j

            @pl.when(g < lastg)
            def _slot():
                wait_in(j)

                @pl.when(q > 0)
                def _wo():
                    wait_out(j)

                compute(j)
                fire_out(g, j)

                @pl.when(g + NB < lastg)
                def _refill():
                    fire_in(g + NB, j)

    for j in range(NB):
        wait_out(j)

    @pl.when(wid == 0)
    def _tail():
        pltpu.sync_copy(tail, tvbuf)
        pltpu.sync_copy(tvbuf, out.at[pl.ds(TG * GW * D, TAILR * D)])


def _fm_body(x_r, emb, fc, wv, bv, out, idx2, rows2, fcv2, outv, tbuf, pv, sems):
    wid = lax.axis_index("c") * 16 + lax.axis_index("s")

    pltpu.sync_copy(wv, pv.at[0])
    pltpu.sync_copy(bv, pv.at[1])

    def fire(g, b):
        row0 = (wid * NCH + g) * NG
        pltpu.sync_copy(x_r.at[pl.ds(row0, NG)], idx2.at[b])
        for r in range(NG):
            pltpu.async_copy(emb.at[idx2.at[b, r]],
                             rows2.at[b, pl.ds(r * IW, IW)], sems.at[b])
            pltpu.async_copy(fc.at[idx2.at[b, r]],
                             fcv2.at[b, pl.ds(r * IW, IW)], sems.at[b])

    def drain(b):
        pltpu.make_async_copy(emb.at[pl.ds(0, RPC)], rows2.at[b],
                              sems.at[b]).wait()
        pltpu.make_async_copy(fc.at[pl.ds(0, RPC)],
                              fcv2.at[b, pl.ds(0, RPC)], sems.at[b]).wait()

    lane = lax.iota(jnp.int32, 16)
    lane16 = lane * 16
    mask10 = (lane < (F - 16)).astype(jnp.float32)

    def compute(g, b):
        wvec = pv[0, :]
        bvec = pv[1, :]
        for grp in range(C // 16):
            @plsc.parallel_loop(0, 16)
            def _elem(i):
                e = grp * 16 + i
                r0 = e * F
                s = jnp.zeros((16,), jnp.float32)
                q = jnp.zeros((16,), jnp.float32)
                for f in range(F):
                    v = rows2[b, r0 + f, :]
                    s = s + v
                    q = q + v * v
                f1 = fcv2[b, pl.ds(r0, 16)]
                f2 = fcv2[b, pl.ds(r0 + 16, 16)]
                t = 0.5 * (s * s - q) + wvec * (f1 + f2 * mask10)
                tbuf[pl.ds(i * 16, 16)] = t

            acc = jnp.zeros((16,), jnp.float32)
            for dcol in range(16):
                acc = acc + plsc.load_gather(tbuf, [lane16 + dcol])
            tot = acc + bvec
            outv[pl.ds(grp * 16, 16)] = 1.0 / (1.0 + jnp.exp(-tot))
        pltpu.sync_copy(outv, out.at[pl.ds(wid * E + g * C, C)])

    fire(0, 0)

    @pl.loop(0, NCH, step=2)
    def _chunks(gg):
        fire(gg + 1, 1)
        drain(0)
        compute(gg, 0)

        @pl.when(gg + 2 < NCH)
        def _refill():
            fire(gg + 2, 0)

        drain(1)
        compute(gg + 1, 1)


def kernel(x, emb_table, fc_table, lin_w, lin_b):
    mesh = plsc.VectorSubcoreMesh(core_axis_name="c", subcore_axis_name="s")

    emb_t = emb_table.T
    tail = emb_table[TG * GW:, :].reshape(-1)
    emb_lin = pl.kernel(
        _tr_body,
        out_type=jax.ShapeDtypeStruct((NR * D,), jnp.float32),
        mesh=mesh,
        compiler_params=pltpu.CompilerParams(needs_layout_passes=False,
                                             use_tc_tiling_on_sc=True),
        scratch_types=(
            [pltpu.VMEM((D, GW), jnp.float32) for _ in range(NB)]
            + [pltpu.VMEM((GW * D,), jnp.float32) for _ in range(NB)]
            + [pltpu.VMEM((TAILR * D,), jnp.float32),
               pltpu.SemaphoreType.DMA((2 * NB,))]
        ),
    )(emb_t, tail)
    emb2 = emb_lin.reshape(NR, D)

    x_r = x.astype(jnp.int32).reshape(B * F // IW, IW)
    fc_flat = fc_table.reshape(-1)
    wv = jnp.broadcast_to(lin_w.reshape(()), (16,)).astype(jnp.float32)
    bv = jnp.broadcast_to(lin_b.reshape(()), (16,)).astype(jnp.float32)

    out = pl.kernel(
        _fm_body,
        out_type=jax.ShapeDtypeStruct((B,), jnp.float32),
        mesh=mesh,
        compiler_params=pltpu.CompilerParams(needs_layout_passes=False,
                                             use_tc_tiling_on_sc=False),
        scratch_types=[
            pltpu.VMEM((2, NG, IW), jnp.int32),
            pltpu.VMEM((2, RPC, D), jnp.float32),
            pltpu.VMEM((2, RPC + 16), jnp.float32),
            pltpu.VMEM((C,), jnp.float32),
            pltpu.VMEM((256,), jnp.float32),
            pltpu.VMEM((2, 16), jnp.float32),
            pltpu.SemaphoreType.DMA((2,)),
        ],
    )(x_r, emb2, fc_flat, wv, bv)
    return out.reshape(B, 1)
